# Initial kernel scaffold; baseline (speedup 1.0000x reference)
import jax
import jax.numpy as jnp
from jax.experimental import pallas as pl


def kernel(x_pf, batch_pf, W1, b1, W2, b2, Wc, bc, Wo1, bo1, Wo2, bo2, Wo3, bo3):
    raise NotImplementedError("write your pallas kernel here")


# speedup vs baseline: 16.5342x; 16.5342x over previous
import jax
import jax.numpy as jnp
from jax import lax
from jax.experimental import pallas as pl
from jax.experimental.pallas import tpu as pltpu

N = 16384
FIN = 4
F = 16
K = 8
NB = 64
R = 256
C = 256
G = N // R
ER = 1024

INF = float('inf')
BIG = 1e9


def _elu(x):
    return jnp.where(x > 0, x, jnp.exp(x) - 1.0)


def _dot_t(a, b):
    return lax.dot_general(a, b, (((1,), (1,)), ((), ())),
                           preferred_element_type=jnp.float32)


def _encode_kernel(x_ref, w1_ref, b1_ref, w2_ref, b2_ref, h_ref, sq_ref):
    h1 = _elu(_dot_t(x_ref[...], w1_ref[...]) + b1_ref[...])
    h = _elu(_dot_t(h1, w2_ref[...]) + b2_ref[...])
    h_ref[...] = h
    sq_ref[...] = jnp.sum(h * h, axis=1, keepdims=True)


def _knn_kernel(cs_ref, cn_ref, h_ref, sq_ref, sqt_ref, b_ref, bt_ref,
                wc_ref, bc_ref, sums_ref, cnts_ref):
    g = pl.program_id(0)
    r0 = g * R
    h_i = h_ref[pl.ds(r0, R), :]
    sq_i = sq_ref[pl.ds(r0, R), :]
    b_i = b_ref[pl.ds(r0, R), :]

    iota_c = lax.broadcasted_iota(jnp.int32, (1, C), 1).astype(jnp.float32)
    iota_k = lax.broadcasted_iota(jnp.int32, (1, K), 1).astype(jnp.float32)

    def merge(state, d2, h_c):
        best_d, best_f = state
        sd = best_d
        new_d, new_f = [], []
        for _t in range(K):
            m_c = jnp.min(d2, axis=1, keepdims=True)
            j_c = jnp.min(jnp.where(d2 == m_c, iota_c, 1e9),
                          axis=1, keepdims=True)
            m_s = jnp.min(sd, axis=1, keepdims=True)
            j_s = jnp.min(jnp.where(sd == m_s, iota_k, 1e9),
                          axis=1, keepdims=True)
            use_c = m_c < m_s
            new_d.append(jnp.where(use_c, m_c, m_s))
            oh_c = jnp.where((iota_c == j_c) & use_c, 1.0, 0.0)
            f_t = lax.dot_general(oh_c, h_c, (((1,), (0,)), ((), ())),
                                  preferred_element_type=jnp.float32)
            for s in range(K):
                f_t = f_t + jnp.where((j_s == s) & (~use_c), best_f[s], 0.0)
            new_f.append(f_t)
            d2 = jnp.where(oh_c > 0, INF, d2)
            sd = jnp.where((iota_k == j_s) & (~use_c), INF, sd)
        return jnp.concatenate(new_d, axis=1), tuple(new_f)

    def chunk_d2(col0):
        h_c = h_ref[pl.ds(col0, C), :]
        sq_c = sqt_ref[:, pl.ds(col0, C)]
        b_c = bt_ref[:, pl.ds(col0, C)]
        d2 = sq_i + sq_c - 2.0 * _dot_t(h_i, h_c)
        return d2, (b_i == b_c), h_c

    state = (jnp.full((R, K), INF, jnp.float32),
             tuple(jnp.zeros((R, F), jnp.float32) for _ in range(K)))

    _d2u, same0, h_c0 = chunk_d2(0)
    d2_bf = jnp.where(same0, INF, BIG + iota_c)
    state = merge(state, d2_bf, h_c0)

    cs = cs_ref[g]

    def body(j, st):
        col0 = (cs + j) * C
        d2, same, h_c = chunk_d2(col0)
        return merge(st, jnp.where(same, d2, INF), h_c)

    state = lax.fori_loop(0, cn_ref[g], body, state)
    _best_d, best_f = state

    a = _dot_t(h_i, wc_ref[:, :F]) + bc_ref[...]
    feats = None
    for s in range(K):
        msg = _elu(a + _dot_t(best_f[s] - h_i, wc_ref[:, F:]))
        feats = msg if feats is None else jnp.maximum(feats, msg)

    iota_b = lax.broadcasted_iota(jnp.int32, (1, NB), 1)
    ohb = jnp.where(b_i == iota_b, 1.0, 0.0)
    part_sums = lax.dot_general(ohb, feats, (((0,), (0,)), ((), ())),
                                preferred_element_type=jnp.float32)
    part_cnts = lax.dot_general(ohb, jnp.ones((R, 1), jnp.float32),
                                (((0,), (0,)), ((), ())),
                                preferred_element_type=jnp.float32)

    @pl.when(g == 0)
    def _init():
        sums_ref[...] = jnp.zeros_like(sums_ref)
        cnts_ref[...] = jnp.zeros_like(cnts_ref)

    sums_ref[...] += part_sums
    cnts_ref[...] += part_cnts


def _head_kernel(sums_ref, cnts_ref, perm_ref, wo1_ref, bo1_ref,
                 wo2_ref, bo2_ref, wo3_ref, bo3_ref, out_ref):
    pooled_v = sums_ref[...] / cnts_ref[...]
    pooled = lax.dot_general(perm_ref[...], pooled_v, (((1,), (0,)), ((), ())),
                             preferred_element_type=jnp.float32)
    o1 = _elu(_dot_t(pooled, wo1_ref[...]) + bo1_ref[...])
    o2 = _elu(_dot_t(o1, wo2_ref[...]) + bo2_ref[...])
    out_ref[...] = _dot_t(o2, wo3_ref[...]) + bo3_ref[...]


def kernel(x_pf, batch_pf, W1, b1, W2, b2, Wc, bc, Wo1, bo1, Wo2, bo2,
           Wo3, bo3):
    batch = batch_pf.astype(jnp.int32)

    h, sq = pl.pallas_call(
        _encode_kernel,
        grid=(N // ER,),
        in_specs=[
            pl.BlockSpec((ER, FIN), lambda i: (i, 0)),
            pl.BlockSpec((F, FIN), lambda i: (0, 0)),
            pl.BlockSpec((1, F), lambda i: (0, 0)),
            pl.BlockSpec((F, F), lambda i: (0, 0)),
            pl.BlockSpec((1, F), lambda i: (0, 0)),
        ],
        out_specs=[
            pl.BlockSpec((ER, F), lambda i: (i, 0)),
            pl.BlockSpec((ER, 1), lambda i: (i, 0)),
        ],
        out_shape=[
            jax.ShapeDtypeStruct((N, F), jnp.float32),
            jax.ShapeDtypeStruct((N, 1), jnp.float32),
        ],
    )(x_pf, W1, b1.reshape(1, F), W2, b2.reshape(1, F))

    offsets = jnp.searchsorted(batch, jnp.arange(NB + 1, dtype=jnp.int32),
                               side='left').astype(jnp.int32)
    b_first = batch[::R]
    b_last = batch[R - 1::R]
    col_lo = jnp.take(offsets, b_first)
    col_hi = jnp.take(offsets, b_last + 1)
    cs = col_lo // C
    cn = (col_hi + C - 1) // C - cs

    sums, cnts = pl.pallas_call(
        _knn_kernel,
        grid_spec=pltpu.PrefetchScalarGridSpec(
            num_scalar_prefetch=2,
            grid=(G,),
            in_specs=[
                pl.BlockSpec((N, F), lambda g, *_: (0, 0)),
                pl.BlockSpec((N, 1), lambda g, *_: (0, 0)),
                pl.BlockSpec((1, N), lambda g, *_: (0, 0)),
                pl.BlockSpec((N, 1), lambda g, *_: (0, 0)),
                pl.BlockSpec((1, N), lambda g, *_: (0, 0)),
                pl.BlockSpec((F, 2 * F), lambda g, *_: (0, 0)),
                pl.BlockSpec((1, F), lambda g, *_: (0, 0)),
            ],
            out_specs=[
                pl.BlockSpec((NB, F), lambda g, *_: (0, 0)),
                pl.BlockSpec((NB, 1), lambda g, *_: (0, 0)),
            ],
        ),
        out_shape=[
            jax.ShapeDtypeStruct((NB, F), jnp.float32),
            jax.ShapeDtypeStruct((NB, 1), jnp.float32),
        ],
    )(cs, cn, h, sq, sq.reshape(1, N), batch.reshape(N, 1),
      batch.reshape(1, N), Wc, bc.reshape(1, F))

    sizes = offsets[1:] - offsets[:-1]
    present = sizes > 0
    ranks = jnp.cumsum(present.astype(jnp.int32)) - 1
    vals = jnp.arange(NB, dtype=jnp.int32)
    uniq = jnp.zeros((NB,), jnp.int32).at[
        jnp.where(present, ranks, NB)].set(vals, mode='drop')
    perm = ((ranks[None, :] == vals[:, None]) & present[None, :]
            ).astype(jnp.float32)

    out = pl.pallas_call(
        _head_kernel,
        out_shape=jax.ShapeDtypeStruct((NB, 2), jnp.float32),
    )(sums, cnts, perm, Wo1, bo1.reshape(1, 8), Wo2, bo2.reshape(1, 4),
      Wo3, bo3.reshape(1, 2))

    return (out, uniq.astype(batch_pf.dtype))
